# Initial kernel scaffold; baseline (speedup 1.0000x reference)
import functools

import jax
import jax.numpy as jnp
from jax import lax
from jax.experimental import pallas as pl
from jax.experimental.pallas import tpu as pltpu
from jax.experimental.pallas import tpu_sc as plsc

_NUM_CORES = 2
_NUM_SUBCORES = 16
_NUM_WORKERS = _NUM_CORES * _NUM_SUBCORES
_CHUNK = 128


@functools.lru_cache(maxsize=None)
def _make_gather(B: int, V: int, D: int):
    assert B % (_NUM_WORKERS * _CHUNK) == 0
    b_per_w = B // _NUM_WORKERS
    n_chunks = b_per_w // _CHUNK

    mesh = plsc.VectorSubcoreMesh(core_axis_name="c", subcore_axis_name="s")

    @functools.partial(
        pl.kernel,
        out_type=jax.ShapeDtypeStruct((B, D), jnp.float32),
        mesh=mesh,
        scratch_types=[
            pltpu.VMEM((_CHUNK,), jnp.int32),
            pltpu.VMEM((_CHUNK, D), jnp.float32),
            pltpu.SemaphoreType.DMA,
        ],
    )
    def gather(idx_hbm, table_hbm, out_hbm, idx_v, rows_v, sem):
        wid = lax.axis_index("s") * _NUM_CORES + lax.axis_index("c")
        base = wid * b_per_w

        @pl.loop(0, n_chunks)
        def _chunk_body(g):
            off = base + g * _CHUNK
            pltpu.sync_copy(idx_hbm.at[pl.ds(off, _CHUNK)], idx_v)
            pltpu.async_copy(table_hbm.at[idx_v], rows_v, sem).wait()
            pltpu.sync_copy(rows_v, out_hbm.at[pl.ds(off, _CHUNK)])

    return gather


def kernel(idxs, table):
    b, s = idxs.shape
    v, d = table.shape
    flat_idx = idxs.reshape(b * s).astype(jnp.int32)
    out = _make_gather(b * s, v, d)(flat_idx, table)
    return out.reshape(b, s, d)


# speedup vs baseline: 3.7549x; 3.7549x over previous
import functools

import jax
import jax.numpy as jnp
from jax import lax
from jax.experimental import pallas as pl
from jax.experimental.pallas import tpu as pltpu
from jax.experimental.pallas import tpu_sc as plsc

_NUM_CORES = 2
_NUM_SUBCORES = 16
_NUM_WORKERS = _NUM_CORES * _NUM_SUBCORES
_CHUNK = 128


@functools.lru_cache(maxsize=None)
def _make_gather(B: int, V: int, D: int):
    assert B % (_NUM_WORKERS * _CHUNK) == 0
    b_per_w = B // _NUM_WORKERS
    n_chunks = b_per_w // _CHUNK

    mesh = plsc.VectorSubcoreMesh(core_axis_name="c", subcore_axis_name="s")

    @functools.partial(
        pl.kernel,
        out_type=jax.ShapeDtypeStruct((B, D), jnp.float32),
        mesh=mesh,
        scratch_types=[
            pltpu.VMEM((_CHUNK,), jnp.int32),
            pltpu.VMEM((_CHUNK, D), jnp.float32),
            pltpu.SemaphoreType.DMA,
        ],
        compiler_params=pltpu.CompilerParams(use_tc_tiling_on_sc=False),
    )
    def gather(idx_hbm, table_hbm, out_hbm, idx_v, rows_v, sem):
        wid = lax.axis_index("s") * _NUM_CORES + lax.axis_index("c")
        base = wid * b_per_w

        @pl.loop(0, n_chunks)
        def _chunk_body(g):
            off = base + g * _CHUNK
            pltpu.sync_copy(idx_hbm.at[pl.ds(off, _CHUNK)], idx_v)
            pltpu.async_copy(table_hbm.at[idx_v], rows_v, sem).wait()
            pltpu.sync_copy(rows_v, out_hbm.at[pl.ds(off, _CHUNK)])

    return gather


def kernel(idxs, table):
    b, s = idxs.shape
    v, d = table.shape
    flat_idx = idxs.reshape(b * s).astype(jnp.int32)
    out = _make_gather(b * s, v, d)(flat_idx, table)
    return out.reshape(b, s, d)


# speedup vs baseline: 4.6589x; 1.2407x over previous
import functools

import jax
import jax.numpy as jnp
from jax import lax
from jax.experimental import pallas as pl
from jax.experimental.pallas import tpu as pltpu
from jax.experimental.pallas import tpu_sc as plsc

_NUM_CORES = 2
_NUM_SUBCORES = 16
_NUM_WORKERS = _NUM_CORES * _NUM_SUBCORES
_CHUNK = 128
_GROUP = 5
_NBUF = 2


@functools.lru_cache(maxsize=None)
def _make_gather(B: int, V: int, D: int):
    b_per_w = B // _NUM_WORKERS
    n_chunks = b_per_w // _CHUNK
    n_groups = n_chunks // _GROUP
    rows_per_group = _GROUP * _CHUNK
    assert B % (_NUM_WORKERS * _CHUNK) == 0
    assert n_chunks % _GROUP == 0 and n_groups % _NBUF == 0

    mesh = plsc.VectorSubcoreMesh(core_axis_name="c", subcore_axis_name="s")

    @functools.partial(
        pl.kernel,
        out_type=jax.ShapeDtypeStruct((B, D), jnp.float32),
        mesh=mesh,
        scratch_types=[
            pltpu.VMEM((n_chunks, _CHUNK), jnp.int32),
            [pltpu.VMEM((rows_per_group, D), jnp.float32) for _ in range(_NBUF)],
            [pltpu.SemaphoreType.DMA for _ in range(_NBUF)],
            [pltpu.SemaphoreType.DMA for _ in range(_NBUF)],
        ],
        compiler_params=pltpu.CompilerParams(use_tc_tiling_on_sc=False),
    )
    def gather(idx_hbm, table_hbm, out_hbm, idx_all, rows, gsems, ssems):
        wid = lax.axis_index("s") * _NUM_CORES + lax.axis_index("c")
        chunk_base = wid * n_chunks
        row_base = wid * b_per_w

        pltpu.sync_copy(idx_hbm.at[pl.ds(chunk_base, n_chunks)], idx_all)

        def gather_desc(g, b, j):
            return pltpu.make_async_copy(
                table_hbm.at[idx_all.at[g * _GROUP + j]],
                rows[b].at[pl.ds(j * _CHUNK, _CHUNK)],
                gsems[b],
            )

        def store_desc(g, b):
            return pltpu.make_async_copy(
                rows[b],
                out_hbm.at[pl.ds(row_base + g * rows_per_group, rows_per_group)],
                ssems[b],
            )

        def start_group(g, b):
            for j in range(_GROUP):
                gather_desc(g, b, j).start()

        for b in range(_NBUF):
            start_group(b, b)

        @pl.loop(0, n_groups, step=_NBUF)
        def _outer(g0):
            for b in range(_NBUF):
                g = g0 + b
                for j in range(_GROUP):
                    gather_desc(g, b, j).wait()
                store_desc(g, b).start()
                gnext = g + _NBUF

                @pl.when(gnext < n_groups)
                def _refill():
                    store_desc(g, b).wait()
                    start_group(gnext, b)

        for b in range(_NBUF):
            store_desc(n_groups - _NBUF + b, b).wait()

    return gather


def kernel(idxs, table):
    b, s = idxs.shape
    v, d = table.shape
    n = b * s
    flat_idx = idxs.reshape(n // _CHUNK, _CHUNK).astype(jnp.int32)
    out = _make_gather(n, v, d)(flat_idx, table)
    return out.reshape(b, s, d)
